# baseline (speedup 1.0000x reference)
import functools
import jax
import jax.numpy as jnp
from jax.experimental import pallas as pl
from jax.experimental.pallas import tpu as pltpu

N = 50000
E = 800000
BLK = 400


def _mm_body(x_ref, w_ref, b_ref, nd_ref, o_ref):
    y = jnp.dot(x_ref[...], w_ref[...], preferred_element_type=jnp.float32)
    y = y * nd_ref[...] + b_ref[...]
    o_ref[...] = jnp.maximum(y, 0.0)


def _mm_relu(x, w, b, nd):
    n, din = x.shape
    dout = w.shape[1]
    grid = n // BLK
    return pl.pallas_call(
        _mm_body,
        grid=(grid,),
        in_specs=[
            pl.BlockSpec((BLK, din), lambda i: (i, 0)),
            pl.BlockSpec((din, dout), lambda i: (0, 0)),
            pl.BlockSpec((1, dout), lambda i: (0, 0)),
            pl.BlockSpec((BLK, 1), lambda i: (i, 0)),
        ],
        out_specs=pl.BlockSpec((BLK, dout), lambda i: (i, 0)),
        out_shape=jax.ShapeDtypeStruct((n, dout), jnp.float32),
    )(x, w, b.reshape(1, dout), nd.reshape(n, 1))


def kernel(features, edge_index, W1, b1, W2, b2, W3, b3, W4, b4, W5, b5, W6, b6):
    src = edge_index[0]
    dst = edge_index[1]
    ones = jnp.ones((E,), dtype=jnp.float32)
    deg_out = jax.ops.segment_sum(ones, src, num_segments=N)
    deg_in = jax.ops.segment_sum(ones, dst, num_segments=N)
    norm_src = jax.lax.rsqrt(jnp.maximum(deg_out, 1.0))
    norm_dst = jax.lax.rsqrt(jnp.maximum(deg_in, 1.0))

    def agg(x):
        feat = x * norm_src[:, None]
        msg = jnp.take(feat, src, axis=0)
        return jax.ops.segment_sum(msg, dst, num_segments=N)

    x1 = _mm_relu(agg(features), W1, b1, norm_dst)
    x2 = _mm_relu(agg(x1), W2, b2, norm_dst)
    x3 = _mm_relu(agg(x2), W3, b3, norm_dst)
    x4 = _mm_relu(agg(x3), W4, b4, norm_dst)
    x5 = _mm_relu(agg(x4 + x3), W5, b5, norm_dst)
    x6 = _mm_relu(agg(x3 + x4 + x5), W6, b6, norm_dst)
    return x6


# speedup vs baseline: 2.0517x; 2.0517x over previous
import functools

import jax
import jax.numpy as jnp
from jax import lax
from jax.experimental import pallas as pl
from jax.experimental.pallas import tpu as pltpu
from jax.experimental.pallas import tpu_sc as plsc

N = 50000
E = 800000
NC = 2
NS = 16
BLK = 400
ZK = 1000


def _sc_mesh():
    return plsc.VectorSubcoreMesh(
        core_axis_name="c", subcore_axis_name="s", num_cores=NC, num_subcores=NS
    )


def _fill(ref, nrows, ncols, value):
    def body(i, _):
        for q in range(ncols // 16):
            ref[i, pl.ds(q * 16, 16)] = jnp.full((16,), value, jnp.float32)
        return 0
    lax.fori_loop(0, nrows, body, 0)


def _fill1d(ref, n, value):
    def body(i, _):
        ref[pl.ds(i * 16, 16)] = jnp.full((16,), value, jnp.float32)
        return 0
    lax.fori_loop(0, n // 16, body, 0)


def _degrees(edge_flat):
    KI = 128
    CD = 16
    ZB = 200
    nchunk = E // KI
    rounds = (nchunk + NS - 1) // NS
    zrounds = (N // ZB + NS - 1) // NS

    @functools.partial(
        pl.kernel,
        out_type=jax.ShapeDtypeStruct((NC * N, CD), jnp.float32),
        mesh=_sc_mesh(),
        scratch_types=[
            pltpu.VMEM((ZB, CD), jnp.float32),
            pltpu.VMEM((KI, CD), jnp.float32),
            pltpu.VMEM((KI,), jnp.int32),
            pltpu.VMEM((ZB, CD), jnp.float32),
            pltpu.VMEM_SHARED((N, CD), jnp.float32),
        ],
        compiler_params=pltpu.CompilerParams(use_tc_tiling_on_sc=False),
    )
    def k(ei_hbm, out_hbm, zbuf, ones, idx, wb, acc):
        c = lax.axis_index("c")
        w = lax.axis_index("s")
        _fill(zbuf, ZB, CD, 0.0)
        onerow = jnp.where(lax.iota(jnp.int32, CD) == 0, 1.0, 0.0)

        def fill_ones(i, _):
            ones[i, pl.ds(0, CD)] = onerow
            return 0
        lax.fori_loop(0, KI, fill_ones, 0)
        for r in range(zrounds):
            t = w + r * NS
            @pl.when(t < N // ZB)
            def _():
                pltpu.sync_copy(zbuf, acc.at[pl.ds(t * ZB, ZB)])
        plsc.subcore_barrier()

        def chunk(r, _):
            t = r * NS + w
            @pl.when(t < nchunk)
            def _():
                eb = c * E + t * KI
                pltpu.sync_copy(ei_hbm.at[pl.ds(eb, KI)], idx)
                pltpu.sync_copy(ones, acc.at[idx], add=True)
            return 0
        lax.fori_loop(0, rounds, chunk, 0)
        plsc.subcore_barrier()
        for r in range(zrounds):
            t = w + r * NS
            @pl.when(t < N // ZB)
            def _():
                pltpu.sync_copy(acc.at[pl.ds(t * ZB, ZB)], wb)
                pltpu.sync_copy(wb, out_hbm.at[pl.ds(c * N + t * ZB, ZB)])

    return k(edge_flat)[:, 0].reshape(NC, N)


def _aggregate(feat_slab, src, dst, S, C):
    KI = 128
    ZB = 200
    split_edges = S == 1
    nchunk = (E // NC if split_edges else E) // KI
    rounds = (nchunk + NS - 1) // NS
    n_out = 2 if split_edges else S
    zrounds = (N // ZB + NS - 1) // NS

    @functools.partial(
        pl.kernel,
        out_type=jax.ShapeDtypeStruct((n_out, N, C), jnp.float32),
        mesh=_sc_mesh(),
        scratch_types=[
            pltpu.VMEM((ZB, C), jnp.float32),
            pltpu.VMEM((ZB, C), jnp.float32),
            pltpu.VMEM((KI,), jnp.int32),
            pltpu.VMEM((KI,), jnp.int32),
            pltpu.VMEM((KI, C), jnp.float32),
            pltpu.VMEM_SHARED((N, C), jnp.float32),
            pltpu.SemaphoreType.DMA,
        ],
        compiler_params=pltpu.CompilerParams(use_tc_tiling_on_sc=False),
    )
    def k(feat_hbm, src_hbm, dst_hbm, out_hbm, zbuf, wb, sidx, didx, rows,
          acc, sem):
        c = lax.axis_index("c")
        w = lax.axis_index("s")
        _fill(zbuf, ZB, C, 0.0)
        for half in range(S // NC if not split_edges else 1):
            for r in range(zrounds):
                t = w + r * NS
                @pl.when(t < N // ZB)
                def _():
                    pltpu.sync_copy(zbuf, acc.at[pl.ds(t * ZB, ZB)])
            plsc.subcore_barrier()

            if split_edges:
                sl = 0
                out_sl = c
                ebase = c * (E // NC)
            else:
                sl = c * (S // NC) + half
                out_sl = sl
                ebase = 0

            def chunk(r, _):
                t = r * NS + w
                @pl.when(t < nchunk)
                def _():
                    eb = ebase + t * KI
                    pltpu.sync_copy(src_hbm.at[pl.ds(eb, KI)], sidx)
                    pltpu.sync_copy(dst_hbm.at[pl.ds(eb, KI)], didx)
                    pltpu.async_copy(feat_hbm.at[sl].at[sidx], rows,
                                     sem).wait()
                    pltpu.sync_copy(rows, acc.at[didx], add=True)
                return 0
            lax.fori_loop(0, rounds, chunk, 0)
            plsc.subcore_barrier()
            for r in range(zrounds):
                t = w + r * NS
                @pl.when(t < N // ZB)
                def _():
                    pltpu.sync_copy(acc.at[pl.ds(t * ZB, ZB)], wb)
                    pltpu.sync_copy(wb, out_hbm.at[out_sl, pl.ds(t * ZB, ZB)])
            plsc.subcore_barrier()

    return k(feat_slab, src, dst)


def _prep_feat1(fpad, deg_out_col):
    def body(x_ref, d_ref, o_ref):
        ns = jax.lax.rsqrt(jnp.maximum(d_ref[...], 1.0))
        o_ref[0] = x_ref[...] * ns

    return pl.pallas_call(
        body,
        grid=(N // BLK,),
        in_specs=[
            pl.BlockSpec((BLK, 16), lambda i: (i, 0)),
            pl.BlockSpec((BLK, 1), lambda i: (i, 0)),
        ],
        out_specs=pl.BlockSpec((1, BLK, 16), lambda i: (0, i, 0)),
        out_shape=jax.ShapeDtypeStruct((1, N, 16), jnp.float32),
    )(fpad, deg_out_col)


def _mm(parts, W, b, deg_in_col, deg_out_col, skips, c_out, sum_parts=False):
    P, _, C_in = parts.shape
    d_out = W.shape[1]
    emit = c_out is not None
    s_out = (d_out // c_out) if emit else 0

    def body(*refs):
        if emit:
            p_ref, w_ref, b_ref, di_ref, do_ref = refs[:5]
            skip_refs = refs[5:5 + len(skips)]
            h_ref, f_ref = refs[5 + len(skips):]
        else:
            p_ref, w_ref, b_ref, di_ref = refs[:4]
            h_ref = refs[4]
        acc = jnp.zeros((BLK, d_out), jnp.float32)
        if sum_parts:
            xs = p_ref[0] + p_ref[1]
            acc = acc + jnp.dot(xs, w_ref[...],
                                preferred_element_type=jnp.float32)
        else:
            for s in range(P):
                acc = acc + jnp.dot(p_ref[s], w_ref[pl.ds(s * C_in, C_in), :],
                                    preferred_element_type=jnp.float32)
        nd = jax.lax.rsqrt(jnp.maximum(di_ref[...], 1.0))
        h = jnp.maximum(acc * nd + b_ref[...], 0.0)
        h_ref[...] = h
        if emit:
            x = h
            for sk in skip_refs:
                x = x + sk[...]
            ns = jax.lax.rsqrt(jnp.maximum(do_ref[...], 1.0))
            f = x * ns
            for s2 in range(s_out):
                f_ref[s2] = f[:, s2 * c_out:(s2 + 1) * c_out]

    in_specs = [
        pl.BlockSpec((P, BLK, C_in), lambda i: (0, i, 0)),
        pl.BlockSpec(W.shape, lambda i: (0, 0)),
        pl.BlockSpec((1, d_out), lambda i: (0, 0)),
        pl.BlockSpec((BLK, 1), lambda i: (i, 0)),
    ]
    if emit:
        in_specs.append(pl.BlockSpec((BLK, 1), lambda i: (i, 0)))
        for _ in skips:
            in_specs.append(pl.BlockSpec((BLK, d_out), lambda i: (i, 0)))
        out_specs = [
            pl.BlockSpec((BLK, d_out), lambda i: (i, 0)),
            pl.BlockSpec((s_out, BLK, c_out), lambda i: (0, i, 0)),
        ]
        out_shape = [
            jax.ShapeDtypeStruct((N, d_out), jnp.float32),
            jax.ShapeDtypeStruct((s_out, N, c_out), jnp.float32),
        ]
        args = (parts, W, b.reshape(1, d_out), deg_in_col, deg_out_col, *skips)
    else:
        out_specs = pl.BlockSpec((BLK, d_out), lambda i: (i, 0))
        out_shape = jax.ShapeDtypeStruct((N, d_out), jnp.float32)
        args = (parts, W, b.reshape(1, d_out), deg_in_col)

    return pl.pallas_call(
        body,
        grid=(N // BLK,),
        in_specs=in_specs,
        out_specs=out_specs,
        out_shape=out_shape,
    )(*args)


_PROBE = 0


def _jnp_ref(features, src, dst, deg, Ws, bs):
    norm_src = jax.lax.rsqrt(jnp.maximum(deg[0], 1.0))
    norm_dst = jax.lax.rsqrt(jnp.maximum(deg[1], 1.0))

    def gc(x, W, b):
        feat = x * norm_src[:, None]
        msg = jnp.take(feat, src, axis=0)
        agg = jax.ops.segment_sum(msg, dst, num_segments=N)
        return jax.nn.relu(agg * norm_dst[:, None] @ W + b)

    x1 = gc(features, Ws[0], bs[0])
    x2 = gc(x1, Ws[1], bs[1])
    x3 = gc(x2, Ws[2], bs[2])
    x4 = gc(x3, Ws[3], bs[3])
    x5 = gc(x4 + x3, Ws[4], bs[4])
    return gc(x3 + x4 + x5, Ws[5], bs[5])


def kernel(features, edge_index, W1, b1, W2, b2, W3, b3, W4, b4, W5, b5, W6, b6):
    src = edge_index[0]
    dst = edge_index[1]

    if _PROBE == 1:
        deg_sc = _degrees(edge_index.reshape(2 * E))
        degj = jnp.stack([
            jax.ops.segment_sum(jnp.ones((E,), jnp.float32), src, num_segments=N),
            jax.ops.segment_sum(jnp.ones((E,), jnp.float32), dst, num_segments=N),
        ])
        base = _jnp_ref(features, src, dst, degj,
                        (W1, W2, W3, W4, W5, W6), (b1, b2, b3, b4, b5, b6))
        return base + 1e6 * jnp.mean(jnp.abs(deg_sc - degj))
    if _PROBE == 2:
        degj = jnp.stack([
            jax.ops.segment_sum(jnp.ones((E,), jnp.float32), src, num_segments=N),
            jax.ops.segment_sum(jnp.ones((E,), jnp.float32), dst, num_segments=N),
        ])
        ns = jax.lax.rsqrt(jnp.maximum(degj[0], 1.0))
        nd = jax.lax.rsqrt(jnp.maximum(degj[1], 1.0))

        def gc(x, W, b):
            feat = x * ns[:, None]
            agg = jax.ops.segment_sum(jnp.take(feat, src, axis=0), dst,
                                      num_segments=N)
            return jax.nn.relu(agg * nd[:, None] @ W + b)

        x3j = gc(gc(gc(features, W1, b1), W2, b2), W3, b3)
        feat = (x3j * ns[:, None]).reshape(N, 8, 32).transpose(1, 0, 2)
        p = _aggregate(feat, src, dst, 8, 32)
        aggj = jax.ops.segment_sum(jnp.take(x3j * ns[:, None], src, axis=0),
                                   dst, num_segments=N)
        aggsc = p.transpose(1, 0, 2).reshape(N, 256)
        base = _jnp_ref(features, src, dst, degj,
                        (W1, W2, W3, W4, W5, W6), (b1, b2, b3, b4, b5, b6))
        return base + 1e4 * jnp.mean(jnp.abs(aggsc - aggj))

    deg = _degrees(edge_index.reshape(2 * E))
    deg_out_col = deg[0].reshape(N, 1)
    deg_in_col = deg[1].reshape(N, 1)

    fpad = jnp.pad(features, ((0, 0), (0, 13)))
    W1p = jnp.pad(W1, ((0, 13), (0, 0)))

    feat1 = _prep_feat1(fpad, deg_out_col)
    p1 = _aggregate(feat1, src, dst, 1, 16)
    h1, feat2 = _mm(p1, W1p, b1, deg_in_col, deg_out_col, (), 32,
                    sum_parts=True)
    p2 = _aggregate(feat2, src, dst, 2, 32)
    h2, feat3 = _mm(p2, W2, b2, deg_in_col, deg_out_col, (), 32)
    p3 = _aggregate(feat3, src, dst, 4, 32)
    h3, feat4 = _mm(p3, W3, b3, deg_in_col, deg_out_col, (), 32)
    p4 = _aggregate(feat4, src, dst, 8, 32)
    h4, feat5 = _mm(p4, W4, b4, deg_in_col, deg_out_col, (h3,), 32)
    p5 = _aggregate(feat5, src, dst, 8, 32)
    h5, feat6 = _mm(p5, W5, b5, deg_in_col, deg_out_col, (h3, h4), 32)
    p6 = _aggregate(feat6, src, dst, 8, 32)
    h6 = _mm(p6, W6, b6, deg_in_col, None, (), None)
    return h6


# speedup vs baseline: 4.2990x; 2.0953x over previous
import functools

import jax
import jax.numpy as jnp
from jax import lax
from jax.experimental import pallas as pl
from jax.experimental.pallas import tpu as pltpu
from jax.experimental.pallas import tpu_sc as plsc

N = 50000
E = 800000
NC = 2
NS = 16
BLK = 400
ZK = 1000


def _sc_mesh():
    return plsc.VectorSubcoreMesh(
        core_axis_name="c", subcore_axis_name="s", num_cores=NC, num_subcores=NS
    )


def _fill(ref, nrows, ncols, value):
    def body(i, _):
        for q in range(ncols // 16):
            ref[i, pl.ds(q * 16, 16)] = jnp.full((16,), value, jnp.float32)
        return 0
    lax.fori_loop(0, nrows, body, 0)


def _fill1d(ref, n, value):
    def body(i, _):
        ref[pl.ds(i * 16, 16)] = jnp.full((16,), value, jnp.float32)
        return 0
    lax.fori_loop(0, n // 16, body, 0)


def _degrees(edge_flat):
    KI = 128
    CD = 16
    ZB = 200
    nchunk = E // KI
    rounds = (nchunk + NS - 1) // NS
    zrounds = (N // ZB + NS - 1) // NS

    @functools.partial(
        pl.kernel,
        out_type=jax.ShapeDtypeStruct((NC * N, CD), jnp.float32),
        mesh=_sc_mesh(),
        scratch_types=[
            pltpu.VMEM((ZB, CD), jnp.float32),
            pltpu.VMEM((KI, CD), jnp.float32),
            pltpu.VMEM((KI,), jnp.int32),
            pltpu.VMEM((ZB, CD), jnp.float32),
            pltpu.VMEM_SHARED((N, CD), jnp.float32),
        ],
        compiler_params=pltpu.CompilerParams(use_tc_tiling_on_sc=False),
    )
    def k(ei_hbm, out_hbm, zbuf, ones, idx, wb, acc):
        c = lax.axis_index("c")
        w = lax.axis_index("s")
        _fill(zbuf, ZB, CD, 0.0)
        onerow = jnp.where(lax.iota(jnp.int32, CD) == 0, 1.0, 0.0)

        def fill_ones(i, _):
            ones[i, pl.ds(0, CD)] = onerow
            return 0
        lax.fori_loop(0, KI, fill_ones, 0)
        for r in range(zrounds):
            t = w + r * NS
            @pl.when(t < N // ZB)
            def _():
                pltpu.sync_copy(zbuf, acc.at[pl.ds(t * ZB, ZB)])
        plsc.subcore_barrier()

        def chunk(r, _):
            t = r * NS + w
            @pl.when(t < nchunk)
            def _():
                eb = c * E + t * KI
                pltpu.sync_copy(ei_hbm.at[pl.ds(eb, KI)], idx)
                pltpu.sync_copy(ones, acc.at[idx], add=True)
            return 0
        lax.fori_loop(0, rounds, chunk, 0)
        plsc.subcore_barrier()
        for r in range(zrounds):
            t = w + r * NS
            @pl.when(t < N // ZB)
            def _():
                pltpu.sync_copy(acc.at[pl.ds(t * ZB, ZB)], wb)
                pltpu.sync_copy(wb, out_hbm.at[pl.ds(c * N + t * ZB, ZB)])

    return k(edge_flat)[:, 0].reshape(NC, N)


def _aggregate(feat_slab, src, dst, S, C):
    KI = 128
    SU = 5
    ZB = 200
    split_edges = S == 1
    nsuper = (E // NC if split_edges else E) // (KI * SU)
    rounds = (nsuper + NS - 1) // NS
    n_out = 2 if split_edges else S
    zrounds = (N // ZB + NS - 1) // NS

    @functools.partial(
        pl.kernel,
        out_type=jax.ShapeDtypeStruct((n_out, N, C), jnp.float32),
        mesh=_sc_mesh(),
        scratch_types=[
            pltpu.VMEM((ZB, C), jnp.float32),
            pltpu.VMEM((SU, KI), jnp.int32),
            pltpu.VMEM((SU, KI), jnp.int32),
            pltpu.VMEM((SU * KI, C), jnp.float32),
            pltpu.VMEM_SHARED((N, C), jnp.float32),
            pltpu.SemaphoreType.DMA,
            pltpu.SemaphoreType.DMA,
        ],
        compiler_params=pltpu.CompilerParams(use_tc_tiling_on_sc=False),
    )
    def k(feat_hbm, src2_hbm, dst2_hbm, out_hbm, zbuf, sidx2, didx2, rows,
          acc, gsem, ssem):
        c = lax.axis_index("c")
        w = lax.axis_index("s")
        _fill(zbuf, ZB, C, 0.0)
        for half in range(S // NC if not split_edges else 1):
            for r in range(zrounds):
                t = w + r * NS
                @pl.when(t < N // ZB)
                def _():
                    pltpu.sync_copy(zbuf, acc.at[pl.ds(t * ZB, ZB)])
            plsc.subcore_barrier()

            if split_edges:
                sl = 0
                out_sl = c
                rbase = c * (E // (NC * KI))
            else:
                sl = c * (S // NC) + half
                out_sl = sl
                rbase = 0

            def chunk(r, _):
                t = r * NS + w
                @pl.when(t < nsuper)
                def _():
                    row0 = rbase + t * SU
                    pltpu.sync_copy(src2_hbm.at[pl.ds(row0, SU)], sidx2)
                    pltpu.sync_copy(dst2_hbm.at[pl.ds(row0, SU)], didx2)
                    gd = [
                        pltpu.async_copy(
                            feat_hbm.at[sl].at[sidx2.at[q]],
                            rows.at[pl.ds(q * KI, KI)], gsem)
                        for q in range(SU)
                    ]
                    for d in gd:
                        d.wait()
                    sd = [
                        pltpu.async_copy(
                            rows.at[pl.ds(q * KI, KI)],
                            acc.at[didx2.at[q]], ssem, add=True)
                        for q in range(SU)
                    ]
                    for d in sd:
                        d.wait()
                return 0
            lax.fori_loop(0, rounds, chunk, 0)
            plsc.subcore_barrier()
            for r in range(zrounds):
                t = w + r * NS
                @pl.when(t < N // ZB)
                def _():
                    pltpu.sync_copy(acc.at[pl.ds(t * ZB, ZB)],
                                    rows.at[pl.ds(0, ZB)])
                    pltpu.sync_copy(rows.at[pl.ds(0, ZB)],
                                    out_hbm.at[out_sl, pl.ds(t * ZB, ZB)])
            plsc.subcore_barrier()

    return k(feat_slab, src.reshape(E // KI, KI), dst.reshape(E // KI, KI))


def _prep_feat1(fpad, deg_out_col):
    def body(x_ref, d_ref, o_ref):
        ns = jax.lax.rsqrt(jnp.maximum(d_ref[...], 1.0))
        o_ref[0] = x_ref[...] * ns

    return pl.pallas_call(
        body,
        grid=(N // BLK,),
        in_specs=[
            pl.BlockSpec((BLK, 16), lambda i: (i, 0)),
            pl.BlockSpec((BLK, 1), lambda i: (i, 0)),
        ],
        out_specs=pl.BlockSpec((1, BLK, 16), lambda i: (0, i, 0)),
        out_shape=jax.ShapeDtypeStruct((1, N, 16), jnp.float32),
    )(fpad, deg_out_col)


def _mm(parts, W, b, deg_in_col, deg_out_col, skips, c_out, sum_parts=False):
    P, _, C_in = parts.shape
    d_out = W.shape[1]
    emit = c_out is not None
    s_out = (d_out // c_out) if emit else 0

    def body(*refs):
        if emit:
            p_ref, w_ref, b_ref, di_ref, do_ref = refs[:5]
            skip_refs = refs[5:5 + len(skips)]
            h_ref, f_ref = refs[5 + len(skips):]
        else:
            p_ref, w_ref, b_ref, di_ref = refs[:4]
            h_ref = refs[4]
        acc = jnp.zeros((BLK, d_out), jnp.float32)
        if sum_parts:
            xs = p_ref[0] + p_ref[1]
            acc = acc + jnp.dot(xs, w_ref[...],
                                preferred_element_type=jnp.float32)
        else:
            for s in range(P):
                acc = acc + jnp.dot(p_ref[s], w_ref[pl.ds(s * C_in, C_in), :],
                                    preferred_element_type=jnp.float32)
        nd = jax.lax.rsqrt(jnp.maximum(di_ref[...], 1.0))
        h = jnp.maximum(acc * nd + b_ref[...], 0.0)
        h_ref[...] = h
        if emit:
            x = h
            for sk in skip_refs:
                x = x + sk[...]
            ns = jax.lax.rsqrt(jnp.maximum(do_ref[...], 1.0))
            f = x * ns
            for s2 in range(s_out):
                f_ref[s2] = f[:, s2 * c_out:(s2 + 1) * c_out]

    in_specs = [
        pl.BlockSpec((P, BLK, C_in), lambda i: (0, i, 0)),
        pl.BlockSpec(W.shape, lambda i: (0, 0)),
        pl.BlockSpec((1, d_out), lambda i: (0, 0)),
        pl.BlockSpec((BLK, 1), lambda i: (i, 0)),
    ]
    if emit:
        in_specs.append(pl.BlockSpec((BLK, 1), lambda i: (i, 0)))
        for _ in skips:
            in_specs.append(pl.BlockSpec((BLK, d_out), lambda i: (i, 0)))
        out_specs = [
            pl.BlockSpec((BLK, d_out), lambda i: (i, 0)),
            pl.BlockSpec((s_out, BLK, c_out), lambda i: (0, i, 0)),
        ]
        out_shape = [
            jax.ShapeDtypeStruct((N, d_out), jnp.float32),
            jax.ShapeDtypeStruct((s_out, N, c_out), jnp.float32),
        ]
        args = (parts, W, b.reshape(1, d_out), deg_in_col, deg_out_col, *skips)
    else:
        out_specs = pl.BlockSpec((BLK, d_out), lambda i: (i, 0))
        out_shape = jax.ShapeDtypeStruct((N, d_out), jnp.float32)
        args = (parts, W, b.reshape(1, d_out), deg_in_col)

    return pl.pallas_call(
        body,
        grid=(N // BLK,),
        in_specs=in_specs,
        out_specs=out_specs,
        out_shape=out_shape,
    )(*args)


_PROBE = 0


def _jnp_ref(features, src, dst, deg, Ws, bs):
    norm_src = jax.lax.rsqrt(jnp.maximum(deg[0], 1.0))
    norm_dst = jax.lax.rsqrt(jnp.maximum(deg[1], 1.0))

    def gc(x, W, b):
        feat = x * norm_src[:, None]
        msg = jnp.take(feat, src, axis=0)
        agg = jax.ops.segment_sum(msg, dst, num_segments=N)
        return jax.nn.relu(agg * norm_dst[:, None] @ W + b)

    x1 = gc(features, Ws[0], bs[0])
    x2 = gc(x1, Ws[1], bs[1])
    x3 = gc(x2, Ws[2], bs[2])
    x4 = gc(x3, Ws[3], bs[3])
    x5 = gc(x4 + x3, Ws[4], bs[4])
    return gc(x3 + x4 + x5, Ws[5], bs[5])


def kernel(features, edge_index, W1, b1, W2, b2, W3, b3, W4, b4, W5, b5, W6, b6):
    src = edge_index[0]
    dst = edge_index[1]

    if _PROBE == 1:
        deg_sc = _degrees(edge_index.reshape(2 * E))
        degj = jnp.stack([
            jax.ops.segment_sum(jnp.ones((E,), jnp.float32), src, num_segments=N),
            jax.ops.segment_sum(jnp.ones((E,), jnp.float32), dst, num_segments=N),
        ])
        base = _jnp_ref(features, src, dst, degj,
                        (W1, W2, W3, W4, W5, W6), (b1, b2, b3, b4, b5, b6))
        return base + 1e6 * jnp.mean(jnp.abs(deg_sc - degj))
    if _PROBE == 2:
        degj = jnp.stack([
            jax.ops.segment_sum(jnp.ones((E,), jnp.float32), src, num_segments=N),
            jax.ops.segment_sum(jnp.ones((E,), jnp.float32), dst, num_segments=N),
        ])
        ns = jax.lax.rsqrt(jnp.maximum(degj[0], 1.0))
        nd = jax.lax.rsqrt(jnp.maximum(degj[1], 1.0))

        def gc(x, W, b):
            feat = x * ns[:, None]
            agg = jax.ops.segment_sum(jnp.take(feat, src, axis=0), dst,
                                      num_segments=N)
            return jax.nn.relu(agg * nd[:, None] @ W + b)

        x3j = gc(gc(gc(features, W1, b1), W2, b2), W3, b3)
        feat = (x3j * ns[:, None]).reshape(N, 8, 32).transpose(1, 0, 2)
        p = _aggregate(feat, src, dst, 8, 32)
        aggj = jax.ops.segment_sum(jnp.take(x3j * ns[:, None], src, axis=0),
                                   dst, num_segments=N)
        aggsc = p.transpose(1, 0, 2).reshape(N, 256)
        base = _jnp_ref(features, src, dst, degj,
                        (W1, W2, W3, W4, W5, W6), (b1, b2, b3, b4, b5, b6))
        return base + 1e4 * jnp.mean(jnp.abs(aggsc - aggj))

    deg = _degrees(edge_index.reshape(2 * E))
    deg_out_col = deg[0].reshape(N, 1)
    deg_in_col = deg[1].reshape(N, 1)

    fpad = jnp.pad(features, ((0, 0), (0, 13)))
    W1p = jnp.pad(W1, ((0, 13), (0, 0)))

    feat1 = _prep_feat1(fpad, deg_out_col)
    p1 = _aggregate(feat1, src, dst, 1, 16)
    h1, feat2 = _mm(p1, W1p, b1, deg_in_col, deg_out_col, (), 32,
                    sum_parts=True)
    p2 = _aggregate(feat2, src, dst, 2, 32)
    h2, feat3 = _mm(p2, W2, b2, deg_in_col, deg_out_col, (), 32)
    p3 = _aggregate(feat3, src, dst, 4, 32)
    h3, feat4 = _mm(p3, W3, b3, deg_in_col, deg_out_col, (), 32)
    p4 = _aggregate(feat4, src, dst, 8, 32)
    h4, feat5 = _mm(p4, W4, b4, deg_in_col, deg_out_col, (h3,), 32)
    p5 = _aggregate(feat5, src, dst, 8, 32)
    h5, feat6 = _mm(p5, W5, b5, deg_in_col, deg_out_col, (h3, h4), 32)
    p6 = _aggregate(feat6, src, dst, 8, 32)
    h6 = _mm(p6, W6, b6, deg_in_col, None, (), None)
    return h6
